# baseline (speedup 1.0000x reference)
import functools

import jax
import jax.numpy as jnp
from jax.experimental import pallas as pl
from jax.experimental.pallas import tpu as pltpu

LANE = 128
_VMEM_LIMIT = 48 * 1024 * 1024


def _round_up(x, m):
    return ((x + m - 1) // m) * m


def _pad2d(a, rows, cols):
    if a.shape == (rows, cols):
        return a
    return jnp.pad(a, ((0, rows - a.shape[0]), (0, cols - a.shape[1])))


def _xw_kernel(x_ref, w_ref, o_ref):
    x = x_ref[...].astype(jnp.bfloat16)
    o_ref[...] = jnp.dot(
        x, w_ref[...], preferred_element_type=jnp.float32
    ).astype(jnp.bfloat16)


def _agg_fused_kernel(adj_ref, h_ref, b_ref, w2_ref, o_ref, m_ref, d_ref,
                      *, tm):
    i = pl.program_id(0)
    a = adj_ref[...].astype(jnp.bfloat16)
    m_ref[...] = (a != 0.0).astype(jnp.int8)
    sub = adj_ref[:, pl.ds(i * tm, tm)]
    rows = jax.lax.broadcasted_iota(jnp.int32, (tm, tm), 0)
    cols = jax.lax.broadcasted_iota(jnp.int32, (tm, tm), 1)
    diag = jnp.sum(jnp.where(rows == cols, sub, 0.0), axis=1, keepdims=True)
    d_col = jnp.sqrt(diag)
    d_ref[...] = jnp.broadcast_to(d_col, d_ref.shape)
    h = jnp.dot(a, h_ref[...], preferred_element_type=jnp.float32)
    h = jnp.maximum(h + b_ref[...], 0.0).astype(jnp.bfloat16)
    h2 = jnp.dot(h, w2_ref[...], preferred_element_type=jnp.float32)
    o_ref[...] = (d_col * h2).astype(jnp.bfloat16)


def _agg_out_kernel(m_ref, h_ref, b_ref, d_ref, o_ref):
    s = m_ref[...].astype(jnp.bfloat16)
    y = jnp.dot(s, h_ref[...], preferred_element_type=jnp.float32)
    o_ref[...] = d_ref[:, 0:1] * y + b_ref[...]


def kernel(x, adj_norm, w1, b1, w2, b2):
    n, fin = x.shape
    hidden = w1.shape[1]
    num_classes = w2.shape[1]

    fin_p = _round_up(fin, LANE)
    hid_p = _round_up(hidden, LANE)
    cls_p = _round_up(num_classes, LANE)

    tm = 512
    tm3 = 1024
    n_p = _round_up(n, max(tm, tm3))

    x_p = _pad2d(x.astype(jnp.float32), n_p, fin_p)
    adj_p = _pad2d(adj_norm.astype(jnp.float32), n_p, n_p)
    w1_p = _pad2d(w1, fin_p, hid_p).astype(jnp.bfloat16)
    b1_p = _pad2d(b1.reshape(1, -1), 1, hid_p)
    w2_p = _pad2d(w2, hid_p, cls_p).astype(jnp.bfloat16)
    b2_p = _pad2d(b2.reshape(1, -1), 1, cls_p)

    h1 = pl.pallas_call(
        _xw_kernel,
        out_shape=jax.ShapeDtypeStruct((n_p, hid_p), jnp.bfloat16),
        grid=(n_p // tm3,),
        in_specs=[
            pl.BlockSpec((tm3, fin_p), lambda i: (i, 0)),
            pl.BlockSpec((fin_p, hid_p), lambda i: (0, 0)),
        ],
        out_specs=pl.BlockSpec((tm3, hid_p), lambda i: (i, 0)),
        compiler_params=pltpu.CompilerParams(
            dimension_semantics=("parallel",), vmem_limit_bytes=_VMEM_LIMIT),
    )(x_p, w1_p)

    h2z, mask, d_mat = pl.pallas_call(
        functools.partial(_agg_fused_kernel, tm=tm3),
        out_shape=[
            jax.ShapeDtypeStruct((n_p, cls_p), jnp.bfloat16),
            jax.ShapeDtypeStruct((n_p, n_p), jnp.int8),
            jax.ShapeDtypeStruct((n_p, LANE), jnp.float32),
        ],
        grid=(n_p // tm3,),
        in_specs=[
            pl.BlockSpec((tm3, n_p), lambda i: (i, 0)),
            pl.BlockSpec((n_p, hid_p), lambda i: (0, 0)),
            pl.BlockSpec((1, hid_p), lambda i: (0, 0)),
            pl.BlockSpec((hid_p, cls_p), lambda i: (0, 0)),
        ],
        out_specs=[
            pl.BlockSpec((tm3, cls_p), lambda i: (i, 0)),
            pl.BlockSpec((tm3, n_p), lambda i: (i, 0)),
            pl.BlockSpec((tm3, LANE), lambda i: (i, 0)),
        ],
        compiler_params=pltpu.CompilerParams(
            dimension_semantics=("parallel",), vmem_limit_bytes=_VMEM_LIMIT),
    )(adj_p, h1, b1_p, w2_p)

    out_p = pl.pallas_call(
        _agg_out_kernel,
        out_shape=jax.ShapeDtypeStruct((n_p, cls_p), jnp.float32),
        grid=(n_p // tm3,),
        in_specs=[
            pl.BlockSpec((tm3, n_p), lambda i: (i, 0)),
            pl.BlockSpec((n_p, cls_p), lambda i: (0, 0)),
            pl.BlockSpec((1, cls_p), lambda i: (0, 0)),
            pl.BlockSpec((tm3, LANE), lambda i: (i, 0)),
        ],
        out_specs=pl.BlockSpec((tm3, cls_p), lambda i: (i, 0)),
        compiler_params=pltpu.CompilerParams(
            dimension_semantics=("parallel",), vmem_limit_bytes=_VMEM_LIMIT),
    )(mask, h2z, b2_p, d_mat)

    return out_p[:n, :num_classes]


# speedup vs baseline: 1.0270x; 1.0270x over previous
import functools

import jax
import jax.numpy as jnp
from jax.experimental import pallas as pl
from jax.experimental.pallas import tpu as pltpu

LANE = 128
_VMEM_LIMIT = 48 * 1024 * 1024


def _round_up(x, m):
    return ((x + m - 1) // m) * m


def _pad2d(a, rows, cols):
    if a.shape == (rows, cols):
        return a
    return jnp.pad(a, ((0, rows - a.shape[0]), (0, cols - a.shape[1])))


def _xw_kernel(x_ref, w_ref, o_ref):
    x = x_ref[...].astype(jnp.bfloat16)
    o_ref[...] = jnp.dot(
        x, w_ref[...], preferred_element_type=jnp.float32
    ).astype(jnp.bfloat16)


def _agg_fused_kernel(adj_ref, h_ref, b_ref, w2_ref, o_ref, m_ref, d_ref,
                      *, tm):
    i = pl.program_id(0)
    a = adj_ref[...].astype(jnp.bfloat16)
    m_ref[...] = (a != 0.0).astype(jnp.int8)
    sub = adj_ref[:, pl.ds(i * tm, tm)]
    rows = jax.lax.broadcasted_iota(jnp.int32, (tm, tm), 0)
    cols = jax.lax.broadcasted_iota(jnp.int32, (tm, tm), 1)
    diag = jnp.sum(jnp.where(rows == cols, sub, 0.0), axis=1, keepdims=True)
    d_col = jnp.sqrt(diag)
    d_ref[...] = jnp.broadcast_to(d_col, d_ref.shape)
    h = jnp.dot(a, h_ref[...], preferred_element_type=jnp.float32)
    h = jnp.maximum(h + b_ref[...], 0.0).astype(jnp.bfloat16)
    h2 = jnp.dot(h, w2_ref[...], preferred_element_type=jnp.float32)
    o_ref[...] = (d_col * h2).astype(jnp.bfloat16)


def _agg_out_kernel(m_ref, h_ref, b_ref, d_ref, o_ref):
    s = m_ref[...].astype(jnp.bfloat16)
    y = jnp.dot(s, h_ref[...], preferred_element_type=jnp.float32)
    o_ref[...] = d_ref[:, 0:1] * y + b_ref[...]


def kernel(x, adj_norm, w1, b1, w2, b2):
    n, fin = x.shape
    hidden = w1.shape[1]
    num_classes = w2.shape[1]

    fin_p = _round_up(fin, LANE)
    hid_p = _round_up(hidden, LANE)
    cls_p = _round_up(num_classes, LANE)

    tm = 512
    tm3 = 1024
    n_p = _round_up(n, max(tm, tm3))

    x_p = _pad2d(x.astype(jnp.float32), n_p, fin_p)
    adj_p = _pad2d(adj_norm.astype(jnp.float32), n_p, n_p)
    w1_p = _pad2d(w1, fin_p, hid_p).astype(jnp.bfloat16)
    b1_p = _pad2d(b1.reshape(1, -1), 1, hid_p)
    w2_p = _pad2d(w2, hid_p, cls_p).astype(jnp.bfloat16)
    b2_p = _pad2d(b2.reshape(1, -1), 1, cls_p)

    h1 = pl.pallas_call(
        _xw_kernel,
        out_shape=jax.ShapeDtypeStruct((n_p, hid_p), jnp.bfloat16),
        grid=(n_p // tm3,),
        in_specs=[
            pl.BlockSpec((tm3, fin_p), lambda i: (i, 0)),
            pl.BlockSpec((fin_p, hid_p), lambda i: (0, 0)),
        ],
        out_specs=pl.BlockSpec((tm3, hid_p), lambda i: (i, 0)),
        compiler_params=pltpu.CompilerParams(
            dimension_semantics=("parallel",), vmem_limit_bytes=_VMEM_LIMIT),
    )(x_p, w1_p)

    h2z, mask, d_mat = pl.pallas_call(
        functools.partial(_agg_fused_kernel, tm=tm),
        out_shape=[
            jax.ShapeDtypeStruct((n_p, cls_p), jnp.bfloat16),
            jax.ShapeDtypeStruct((n_p, n_p), jnp.int8),
            jax.ShapeDtypeStruct((n_p, LANE), jnp.float32),
        ],
        grid=(n_p // tm,),
        in_specs=[
            pl.BlockSpec((tm, n_p), lambda i: (i, 0)),
            pl.BlockSpec((n_p, hid_p), lambda i: (0, 0)),
            pl.BlockSpec((1, hid_p), lambda i: (0, 0)),
            pl.BlockSpec((hid_p, cls_p), lambda i: (0, 0)),
        ],
        out_specs=[
            pl.BlockSpec((tm, cls_p), lambda i: (i, 0)),
            pl.BlockSpec((tm, n_p), lambda i: (i, 0)),
            pl.BlockSpec((tm, LANE), lambda i: (i, 0)),
        ],
        compiler_params=pltpu.CompilerParams(
            dimension_semantics=("parallel",), vmem_limit_bytes=_VMEM_LIMIT),
    )(adj_p, h1, b1_p, w2_p)

    out_p = pl.pallas_call(
        _agg_out_kernel,
        out_shape=jax.ShapeDtypeStruct((n_p, cls_p), jnp.float32),
        grid=(n_p // tm3,),
        in_specs=[
            pl.BlockSpec((tm3, n_p), lambda i: (i, 0)),
            pl.BlockSpec((n_p, cls_p), lambda i: (0, 0)),
            pl.BlockSpec((1, cls_p), lambda i: (0, 0)),
            pl.BlockSpec((tm3, LANE), lambda i: (i, 0)),
        ],
        out_specs=pl.BlockSpec((tm3, cls_p), lambda i: (i, 0)),
        compiler_params=pltpu.CompilerParams(
            dimension_semantics=("parallel",), vmem_limit_bytes=_VMEM_LIMIT),
    )(mask, h2z, b2_p, d_mat)

    return out_p[:n, :num_classes]
